# baseline (speedup 1.0000x reference)
import jax
import jax.numpy as jnp
from jax import lax
from jax.experimental import pallas as pl
from jax.experimental.pallas import tpu as pltpu
from jax.experimental.pallas import tpu_sc as plsc

_T = 4096
_D = 128
_V = 76
_BT = 512
_NB = _T // _BT
_NCODE = 64

_NC, _NS = 2, 16
_NW = _NC * _NS
_BW = _T // _NW


def _build_body(ids_ref, q6_ref, table_ref, basis_ref, fused_ref, idx_ref):
    c = lax.broadcasted_iota(jnp.int32, (_NCODE, 6), 0)
    k = lax.broadcasted_iota(jnp.int32, (_NCODE, 6), 1)
    bits = ((c >> k) & 1).astype(jnp.float32)
    geo = jnp.dot(bits, basis_ref[...], preferred_element_type=jnp.float32)
    tab = table_ref[...]
    fused_ref[...] = (tab[:, None, :] + geo[None, :, :]).reshape(
        _V * _NCODE, _D)
    w = (1 << lax.broadcasted_iota(jnp.int32, (1, 6), 1)).astype(jnp.float32)
    code = jnp.sum(q6_ref[...] * w, axis=1).astype(jnp.int32)
    idx_ref[...] = ids_ref[...] * _NCODE + code


def _tc_body(scale_ref, q6_ref, ham_ref):
    i = pl.program_id(0)
    j = pl.program_id(1)
    rows = q6_ref[pl.ds(i * _BT, _BT), :]
    cols = q6_ref[pl.ds(j * _BT, _BT), :]
    g = jax.lax.dot_general(rows, cols, (((1,), (1,)), ((), ())),
                            preferred_element_type=jnp.float32)
    si = jnp.sum(rows, axis=1)
    sj = jnp.sum(cols, axis=1)
    scale = scale_ref[0]
    ham_ref[...] = (-scale) * (si[:, None] + sj[None, :] - 2.0 * g)


def _sc_body(idx_hbm, fused_hbm, out_hbm, idx_ref, rows_ref, sem):
    wid = lax.axis_index("s") * _NC + lax.axis_index("c")
    base = wid * _BW
    pltpu.sync_copy(idx_hbm.at[pl.ds(base, _BW)], idx_ref)
    pltpu.async_copy(fused_hbm.at[idx_ref], rows_ref, sem).wait()
    pltpu.sync_copy(rows_ref, out_hbm.at[pl.ds(base, _BW)])


def kernel(token_ids, q6_vecs, token_embed, q6_basis, hamming_scale):
    scale = jnp.reshape(hamming_scale, (1,)).astype(jnp.float32)
    ids = token_ids.astype(jnp.int32)
    q6f = q6_vecs.astype(jnp.float32)

    fused, idx = pl.pallas_call(
        _build_body,
        in_specs=[
            pl.BlockSpec(memory_space=pltpu.VMEM),
            pl.BlockSpec(memory_space=pltpu.VMEM),
            pl.BlockSpec(memory_space=pltpu.VMEM),
            pl.BlockSpec(memory_space=pltpu.VMEM),
        ],
        out_specs=[
            pl.BlockSpec(memory_space=pltpu.VMEM),
            pl.BlockSpec(memory_space=pltpu.VMEM),
        ],
        out_shape=[
            jax.ShapeDtypeStruct((_V * _NCODE, _D), jnp.float32),
            jax.ShapeDtypeStruct((_T,), jnp.int32),
        ],
    )(ids, q6f, token_embed.astype(jnp.float32), q6_basis.astype(jnp.float32))

    sc_combined = pl.kernel(
        _sc_body,
        out_type=jax.ShapeDtypeStruct((_T, _D), jnp.float32),
        mesh=plsc.VectorSubcoreMesh(
            core_axis_name="c", subcore_axis_name="s",
            num_cores=_NC, num_subcores=_NS),
        scratch_types=[
            pltpu.VMEM((_BW,), jnp.int32),
            pltpu.VMEM((_BW, _D), jnp.float32),
            pltpu.SemaphoreType.DMA,
        ],
    )
    comb = sc_combined(idx, fused)

    ham = pl.pallas_call(
        _tc_body,
        grid=(_NB, _NB),
        in_specs=[
            pl.BlockSpec(memory_space=pltpu.SMEM),
            pl.BlockSpec(memory_space=pltpu.VMEM),
        ],
        out_specs=pl.BlockSpec((_BT, _BT), lambda i, j: (i, j)),
        out_shape=jax.ShapeDtypeStruct((_T, _T), jnp.float32),
        compiler_params=pltpu.CompilerParams(
            dimension_semantics=("arbitrary", "arbitrary")),
    )(scale, q6f)

    return comb[None], ham


# speedup vs baseline: 1.5312x; 1.5312x over previous
import jax
import jax.numpy as jnp
from jax import lax
from jax.experimental import pallas as pl
from jax.experimental.pallas import tpu as pltpu
from jax.experimental.pallas import tpu_sc as plsc

_T = 4096
_D = 128
_V = 76
_BT = 512
_NB = _T // _BT
_NCODE = 64

_NC, _NS, _L = 2, 16, 16
_NW = _NC * _NS
_BW = _T // _NW
_NG = _BW // _L


def _build_body(table_ref, basis_ref, fused_ref):
    c = lax.broadcasted_iota(jnp.int32, (_NCODE, 6), 0)
    k = lax.broadcasted_iota(jnp.int32, (_NCODE, 6), 1)
    bits = ((c >> k) & 1).astype(jnp.float32)
    geo = jnp.dot(bits, basis_ref[...], preferred_element_type=jnp.float32)
    tab = table_ref[...]
    fused_ref[...] = (tab[:, None, :] + geo[None, :, :]).reshape(
        _V * _NCODE, _D)


def _tc_body(scale_ref, q6_ref, ham_ref):
    i = pl.program_id(0)
    rows = q6_ref[pl.ds(i * _BT, _BT), :]
    cols = q6_ref[...]
    g = jax.lax.dot_general(rows, cols, (((1,), (1,)), ((), ())),
                            preferred_element_type=jnp.float32)
    si = jnp.sum(rows, axis=1)
    sj = jnp.sum(cols, axis=1)
    scale = scale_ref[0]
    ham_ref[...] = (-scale) * (si[:, None] + sj[None, :] - 2.0 * g)


def _sc_body(ids_hbm, q6t_hbm, fused_hbm, out_hbm, idx_ref, rows_ref, q6_ref, sem):
    wid = lax.axis_index("s") * _NC + lax.axis_index("c")
    base = wid * _BW
    pltpu.sync_copy(ids_hbm.at[pl.ds(base, _BW)], idx_ref.at[0])
    pltpu.sync_copy(q6t_hbm.at[:, pl.ds(base, _BW)], q6_ref)
    for g in range(_NG):
        ids_v = idx_ref[0, pl.ds(g * _L, _L)]
        code = jnp.zeros((_L,), jnp.int32)
        for k in range(6):
            qv = q6_ref[k, pl.ds(g * _L, _L)]
            code = code + jnp.where(qv > 0.5, jnp.int32(1 << k), jnp.int32(0))
        idx_ref[1, pl.ds(g * _L, _L)] = ids_v * _NCODE + code
    pltpu.async_copy(fused_hbm.at[idx_ref.at[1]], rows_ref, sem).wait()
    pltpu.sync_copy(rows_ref, out_hbm.at[pl.ds(base, _BW)])


def kernel(token_ids, q6_vecs, token_embed, q6_basis, hamming_scale):
    scale = jnp.reshape(hamming_scale, (1,)).astype(jnp.float32)
    ids = token_ids.astype(jnp.int32)
    q6f = q6_vecs.astype(jnp.float32)

    fused = pl.pallas_call(
        _build_body,
        in_specs=[
            pl.BlockSpec(memory_space=pltpu.VMEM),
            pl.BlockSpec(memory_space=pltpu.VMEM),
        ],
        out_specs=pl.BlockSpec(memory_space=pltpu.VMEM),
        out_shape=jax.ShapeDtypeStruct((_V * _NCODE, _D), jnp.float32),
    )(token_embed.astype(jnp.float32), q6_basis.astype(jnp.float32))

    sc_combined = pl.kernel(
        _sc_body,
        out_type=jax.ShapeDtypeStruct((_T, _D), jnp.float32),
        mesh=plsc.VectorSubcoreMesh(
            core_axis_name="c", subcore_axis_name="s",
            num_cores=_NC, num_subcores=_NS),
        scratch_types=[
            pltpu.VMEM((2, _BW), jnp.int32),
            pltpu.VMEM((_BW, _D), jnp.float32),
            pltpu.VMEM((6, _BW), jnp.float32),
            pltpu.SemaphoreType.DMA,
        ],
    )
    comb = sc_combined(ids, q6f.T, fused)

    ham = pl.pallas_call(
        _tc_body,
        grid=(_NB,),
        in_specs=[
            pl.BlockSpec(memory_space=pltpu.SMEM),
            pl.BlockSpec(memory_space=pltpu.VMEM),
        ],
        out_specs=pl.BlockSpec((_BT, _T), lambda i: (i, 0)),
        out_shape=jax.ShapeDtypeStruct((_T, _T), jnp.float32),
        compiler_params=pltpu.CompilerParams(
            dimension_semantics=("arbitrary",)),
    )(scale, q6f)

    return comb[None], ham
